# Initial kernel scaffold; baseline (speedup 1.0000x reference)
import jax
import jax.numpy as jnp
from jax.experimental import pallas as pl


def kernel(x, edge_index, output_size, W1, b1, W2, b2, Wfc, bfc):
    raise NotImplementedError("write your pallas kernel here")


# speedup vs baseline: 51.3043x; 51.3043x over previous
import functools

import jax
import jax.numpy as jnp
from jax import lax
from jax.experimental import pallas as pl
from jax.experimental.pallas import tpu as pltpu
from jax.experimental.pallas import tpu_sc as plsc

_NC = 2
_NS = 16
_NW = _NC * _NS
_B = 80
_H = 16


_CHUNK = 624


def _zero_acc(zero_hbm, acc, s, N):
    pltpu.sync_copy(zero_hbm.at[pl.ds(s * _CHUNK, _CHUNK)],
                    acc.at[pl.ds(s * _CHUNK, _CHUNK)])
    tail = N - _CHUNK * _NS
    if tail:
        @pl.when(s == _NS - 1)
        def _():
            pltpu.sync_copy(zero_hbm.at[pl.ds(_CHUNK * _NS, tail)],
                            acc.at[pl.ds(_CHUNK * _NS, tail)])


def _writeback_acc(acc, out_hbm, c, s, N):
    pltpu.sync_copy(acc.at[pl.ds(s * _CHUNK, _CHUNK)],
                    out_hbm.at[c, pl.ds(s * _CHUNK, _CHUNK)])
    tail = N - _CHUNK * _NS
    if tail:
        @pl.when(s == _NS - 1)
        def _():
            pltpu.sync_copy(acc.at[pl.ds(_CHUNK * _NS, tail)],
                            out_hbm.at[c, pl.ds(_CHUNK * _NS, tail)])


def _make_deg_kernel(N, E):
    ew = E // _NW
    nb = ew // _B
    mesh = plsc.VectorSubcoreMesh(core_axis_name="c", subcore_axis_name="s")

    @functools.partial(
        pl.kernel,
        out_type=jax.ShapeDtypeStruct((_NC, N, _H), jnp.float32),
        mesh=mesh,
        compiler_params=pltpu.CompilerParams(use_tc_tiling_on_sc=False),
        scratch_types=[
            pltpu.VMEM((nb, _B), jnp.int32),
            pltpu.VMEM((_B, _H), jnp.float32),
            pltpu.VMEM_SHARED((N, _H), jnp.float32),
            pltpu.SemaphoreType.DMA,
        ],
    )
    def k(dst_hbm, zero_hbm, out_hbm, dstv, ones_v, acc, sem):
        c = lax.axis_index("c")
        s = lax.axis_index("s")
        w = c * _NS + s
        _zero_acc(zero_hbm, acc, s, N)
        pltpu.sync_copy(dst_hbm.at[w], dstv)

        def mkones(i, carry):
            ones_v[i, :] = jnp.full((_H,), 1.0, jnp.float32)
            return carry
        lax.fori_loop(0, _B, mkones, 0)
        plsc.subcore_barrier()

        def fire(j, carry):
            pltpu.async_copy(ones_v, acc.at[dstv.at[j]], sem, add=True)
            return carry
        lax.fori_loop(0, nb, fire, 0)

        def drain(j, carry):
            pltpu.make_async_copy(ones_v, acc.at[dstv.at[0]], sem).wait()
            return carry
        lax.fori_loop(0, nb, drain, 0)
        plsc.subcore_barrier()
        _writeback_acc(acc, out_hbm, c, s, N)

    return k


def _make_msg_kernel(N, E):
    ew = E // _NW
    nb = ew // _B
    ph = (nb + 1) // 2
    mesh = plsc.VectorSubcoreMesh(core_axis_name="c", subcore_axis_name="s")

    @functools.partial(
        pl.kernel,
        out_type=jax.ShapeDtypeStruct((_NC, N, _H), jnp.float32),
        mesh=mesh,
        compiler_params=pltpu.CompilerParams(use_tc_tiling_on_sc=False),
        scratch_types=[
            pltpu.VMEM((nb, _B), jnp.int32),
            pltpu.VMEM((nb, _B), jnp.int32),
            pltpu.VMEM((ph * _B, _H), jnp.float32),
            pltpu.VMEM_SHARED((N, _H), jnp.float32),
            pltpu.SemaphoreType.DMA,
            pltpu.SemaphoreType.DMA,
        ],
    )
    def k(g_hbm, src_hbm, dst_hbm, zero_hbm, out_hbm,
          srcv, dstv, rows, acc, gsem, ssem):
        c = lax.axis_index("c")
        s = lax.axis_index("s")
        w = c * _NS + s
        _zero_acc(zero_hbm, acc, s, N)
        pltpu.sync_copy(src_hbm.at[w], srcv)
        pltpu.sync_copy(dst_hbm.at[w], dstv)
        plsc.subcore_barrier()

        for p in range(2):
            off = p * ph
            cnt = ph if p == 0 else nb - ph

            def fire_g(j, carry):
                pltpu.async_copy(g_hbm.at[srcv.at[off + j]],
                                 rows.at[pl.ds(j * _B, _B)], gsem)
                return carry
            lax.fori_loop(0, cnt, fire_g, 0)

            def drain_g(j, carry):
                pltpu.make_async_copy(g_hbm.at[srcv.at[0]],
                                      rows.at[pl.ds(0, _B)], gsem).wait()
                return carry
            lax.fori_loop(0, cnt, drain_g, 0)

            def fire_s(j, carry):
                pltpu.async_copy(rows.at[pl.ds(j * _B, _B)],
                                 acc.at[dstv.at[off + j]], ssem, add=True)
                return carry
            lax.fori_loop(0, cnt, fire_s, 0)

            def drain_s(j, carry):
                pltpu.make_async_copy(rows.at[pl.ds(0, _B)],
                                      acc.at[dstv.at[0]], ssem).wait()
                return carry
            lax.fori_loop(0, cnt, drain_s, 0)

        plsc.subcore_barrier()
        _writeback_acc(acc, out_hbm, c, s, N)

    return k


_BM = 2000


def _tc1(x, W1, d0, d1):
    N, D = x.shape
    H = W1.shape[1]

    def body(x_ref, w_ref, d0_ref, d1_ref, g_ref, dinv_ref):
        deg = d0_ref[...] + d1_ref[...] + 1.0
        dinv = lax.rsqrt(deg)
        a = jnp.dot(x_ref[...], w_ref[...], preferred_element_type=jnp.float32)
        g_ref[...] = a * dinv
        dinv_ref[...] = dinv

    return pl.pallas_call(
        body,
        grid=(N // _BM,),
        in_specs=[
            pl.BlockSpec((_BM, D), lambda i: (i, 0)),
            pl.BlockSpec((D, H), lambda i: (0, 0)),
            pl.BlockSpec((_BM, H), lambda i: (i, 0)),
            pl.BlockSpec((_BM, H), lambda i: (i, 0)),
        ],
        out_specs=[pl.BlockSpec((_BM, H), lambda i: (i, 0))] * 2,
        out_shape=[jax.ShapeDtypeStruct((N, H), jnp.float32)] * 2,
    )(x, W1, d0, d1)


def _tc2(m0, m1, g, dinv, b, W2):
    N, H = g.shape

    def body(m0_ref, m1_ref, g_ref, dinv_ref, b_ref, w_ref, out_ref):
        h = dinv_ref[...] * (m0_ref[...] + m1_ref[...] + g_ref[...]) + b_ref[...]
        h = jnp.maximum(h, 0.0)
        a = jnp.dot(h, w_ref[...], preferred_element_type=jnp.float32)
        out_ref[...] = a * dinv_ref[...]

    return pl.pallas_call(
        body,
        grid=(N // _BM,),
        in_specs=[
            pl.BlockSpec((_BM, H), lambda i: (i, 0)),
            pl.BlockSpec((_BM, H), lambda i: (i, 0)),
            pl.BlockSpec((_BM, H), lambda i: (i, 0)),
            pl.BlockSpec((_BM, H), lambda i: (i, 0)),
            pl.BlockSpec((1, H), lambda i: (0, 0)),
            pl.BlockSpec((H, H), lambda i: (0, 0)),
        ],
        out_specs=pl.BlockSpec((_BM, H), lambda i: (i, 0)),
        out_shape=jax.ShapeDtypeStruct((N, H), jnp.float32),
    )(m0, m1, g, dinv, b, W2)


def _tc3(m0, m1, g, dinv, b, Wfc, bfc):
    N, H = g.shape
    OUT = Wfc.shape[1]

    def body(m0_ref, m1_ref, g_ref, dinv_ref, b_ref, w_ref, bfc_ref, out_ref):
        h = dinv_ref[...] * (m0_ref[...] + m1_ref[...] + g_ref[...]) + b_ref[...]
        h = jnp.maximum(h, 0.0)
        out_ref[...] = (jnp.dot(h, w_ref[...], preferred_element_type=jnp.float32)
                        + bfc_ref[...])

    return pl.pallas_call(
        body,
        grid=(N // _BM,),
        in_specs=[
            pl.BlockSpec((_BM, H), lambda i: (i, 0)),
            pl.BlockSpec((_BM, H), lambda i: (i, 0)),
            pl.BlockSpec((_BM, H), lambda i: (i, 0)),
            pl.BlockSpec((_BM, H), lambda i: (i, 0)),
            pl.BlockSpec((1, H), lambda i: (0, 0)),
            pl.BlockSpec((H, OUT), lambda i: (0, 0)),
            pl.BlockSpec((1, OUT), lambda i: (0, 0)),
        ],
        out_specs=pl.BlockSpec((_BM, OUT), lambda i: (i, 0)),
        out_shape=jax.ShapeDtypeStruct((N, OUT), jnp.float32),
    )(m0, m1, g, dinv, b, Wfc, bfc)


def kernel(x, edge_index, output_size, W1, b1, W2, b2, Wfc, bfc):
    N, D = x.shape
    E = edge_index.shape[1]
    H = W1.shape[1]
    OUT = Wfc.shape[1]
    assert H == _H and E % (_NW * _B) == 0 and N % _BM == 0

    nb = E // (_NW * _B)
    src2 = edge_index[0].astype(jnp.int32).reshape(_NW, nb, _B)
    dst2 = edge_index[1].astype(jnp.int32).reshape(_NW, nb, _B)
    zeros = jnp.zeros((N, H), jnp.float32)

    deg_k = _make_deg_kernel(N, E)
    msg_k = _make_msg_kernel(N, E)

    degp = deg_k(dst2, zeros)
    g1, dinv = _tc1(x, W1, degp[0], degp[1])
    m1 = msg_k(g1, src2, dst2, zeros)
    g2 = _tc2(m1[0], m1[1], g1, dinv, b1.reshape(1, H), W2)
    m2 = msg_k(g2, src2, dst2, zeros)
    return _tc3(m2[0], m2[1], g2, dinv, b2.reshape(1, H), Wfc,
                bfc.reshape(1, OUT))


# speedup vs baseline: 61.4656x; 1.1981x over previous
import functools

import jax
import jax.numpy as jnp
from jax import lax
from jax.experimental import pallas as pl
from jax.experimental.pallas import tpu as pltpu
from jax.experimental.pallas import tpu_sc as plsc

_NC = 2
_NS = 16
_NW = _NC * _NS
_B = 80
_H = 16


_CHUNK = 624


def _zero_acc(zero_hbm, acc, s, N):
    pltpu.sync_copy(zero_hbm.at[pl.ds(s * _CHUNK, _CHUNK)],
                    acc.at[pl.ds(s * _CHUNK, _CHUNK)])
    tail = N - _CHUNK * _NS
    if tail:
        @pl.when(s == _NS - 1)
        def _():
            pltpu.sync_copy(zero_hbm.at[pl.ds(_CHUNK * _NS, tail)],
                            acc.at[pl.ds(_CHUNK * _NS, tail)])


def _writeback_acc(acc, out_hbm, c, s, N):
    pltpu.sync_copy(acc.at[pl.ds(s * _CHUNK, _CHUNK)],
                    out_hbm.at[c, pl.ds(s * _CHUNK, _CHUNK)])
    tail = N - _CHUNK * _NS
    if tail:
        @pl.when(s == _NS - 1)
        def _():
            pltpu.sync_copy(acc.at[pl.ds(_CHUNK * _NS, tail)],
                            out_hbm.at[c, pl.ds(_CHUNK * _NS, tail)])


def _make_deg_kernel(N, E):
    ew = E // _NW
    nb = ew // _B
    mesh = plsc.VectorSubcoreMesh(core_axis_name="c", subcore_axis_name="s")

    @functools.partial(
        pl.kernel,
        out_type=jax.ShapeDtypeStruct((_NC, N, _H), jnp.float32),
        mesh=mesh,
        compiler_params=pltpu.CompilerParams(use_tc_tiling_on_sc=False),
        scratch_types=[
            pltpu.VMEM((ew,), jnp.int32),
            pltpu.VMEM((_B, _H), jnp.float32),
            pltpu.VMEM_SHARED((N, _H), jnp.float32),
            pltpu.SemaphoreType.DMA,
        ],
    )
    def k(edge_hbm, zero_hbm, out_hbm, dstv, ones_v, acc, sem):
        c = lax.axis_index("c")
        s = lax.axis_index("s")
        w = c * _NS + s
        _zero_acc(zero_hbm, acc, s, N)
        pltpu.sync_copy(edge_hbm.at[1, pl.ds(w * ew, ew)], dstv)

        def mkones(i, carry):
            ones_v[i, :] = jnp.full((_H,), 1.0, jnp.float32)
            return carry
        lax.fori_loop(0, _B, mkones, 0)
        plsc.subcore_barrier()

        def fire(j, carry):
            pltpu.async_copy(ones_v, acc.at[dstv.at[pl.ds(j * _B, _B)]],
                             sem, add=True)
            return carry
        lax.fori_loop(0, nb, fire, 0)

        def drain(j, carry):
            pltpu.make_async_copy(ones_v, acc.at[dstv.at[pl.ds(0, _B)]],
                                  sem).wait()
            return carry
        lax.fori_loop(0, nb, drain, 0)
        plsc.subcore_barrier()
        _writeback_acc(acc, out_hbm, c, s, N)

    return k


def _make_msg_kernel(N, E):
    ew = E // _NW
    nb = ew // _B
    ph = (nb + 1) // 2
    mesh = plsc.VectorSubcoreMesh(core_axis_name="c", subcore_axis_name="s")

    @functools.partial(
        pl.kernel,
        out_type=jax.ShapeDtypeStruct((_NC, N, _H), jnp.float32),
        mesh=mesh,
        compiler_params=pltpu.CompilerParams(use_tc_tiling_on_sc=False),
        scratch_types=[
            pltpu.VMEM((ew,), jnp.int32),
            pltpu.VMEM((ew,), jnp.int32),
            pltpu.VMEM((ph * _B, _H), jnp.float32),
            pltpu.VMEM_SHARED((N, _H), jnp.float32),
            pltpu.SemaphoreType.DMA,
            pltpu.SemaphoreType.DMA,
        ],
    )
    def k(g_hbm, edge_hbm, zero_hbm, out_hbm,
          srcv, dstv, rows, acc, gsem, ssem):
        c = lax.axis_index("c")
        s = lax.axis_index("s")
        w = c * _NS + s
        _zero_acc(zero_hbm, acc, s, N)
        pltpu.sync_copy(edge_hbm.at[0, pl.ds(w * ew, ew)], srcv)
        pltpu.sync_copy(edge_hbm.at[1, pl.ds(w * ew, ew)], dstv)
        plsc.subcore_barrier()

        for p in range(2):
            off = p * ph * _B
            cnt = ph if p == 0 else nb - ph

            def fire_g(j, carry):
                pltpu.async_copy(g_hbm.at[srcv.at[pl.ds(off + j * _B, _B)]],
                                 rows.at[pl.ds(j * _B, _B)], gsem)
                return carry
            lax.fori_loop(0, cnt, fire_g, 0)

            def drain_g(j, carry):
                pltpu.make_async_copy(g_hbm.at[srcv.at[pl.ds(0, _B)]],
                                      rows.at[pl.ds(0, _B)], gsem).wait()
                return carry
            lax.fori_loop(0, cnt, drain_g, 0)

            def fire_s(j, carry):
                pltpu.async_copy(rows.at[pl.ds(j * _B, _B)],
                                 acc.at[dstv.at[pl.ds(off + j * _B, _B)]],
                                 ssem, add=True)
                return carry
            lax.fori_loop(0, cnt, fire_s, 0)

            def drain_s(j, carry):
                pltpu.make_async_copy(rows.at[pl.ds(0, _B)],
                                      acc.at[dstv.at[pl.ds(0, _B)]],
                                      ssem).wait()
                return carry
            lax.fori_loop(0, cnt, drain_s, 0)

        plsc.subcore_barrier()
        _writeback_acc(acc, out_hbm, c, s, N)

    return k


_BM = 2000


def _pspec(i_map=None):
    def mk(c):
        return pl.BlockSpec((1, _BM, _H), lambda i, c=c: (c, i, 0))
    return [mk(0), mk(1)]


def _tc1(x, W1, degp):
    N, D = x.shape
    H = W1.shape[1]

    def body(x_ref, w_ref, d0_ref, d1_ref, g_ref, dinv_ref):
        deg = d0_ref[0] + d1_ref[0] + 1.0
        dinv = lax.rsqrt(deg)
        a = jnp.dot(x_ref[...], w_ref[...], preferred_element_type=jnp.float32)
        g_ref[...] = a * dinv
        dinv_ref[...] = dinv

    return pl.pallas_call(
        body,
        grid=(N // _BM,),
        in_specs=[
            pl.BlockSpec((_BM, D), lambda i: (i, 0)),
            pl.BlockSpec((D, H), lambda i: (0, 0)),
            *_pspec(),
        ],
        out_specs=[pl.BlockSpec((_BM, H), lambda i: (i, 0))] * 2,
        out_shape=[jax.ShapeDtypeStruct((N, H), jnp.float32)] * 2,
    )(x, W1, degp, degp)


def _tc2(mp, g, dinv, b, W2):
    N, H = g.shape

    def body(m0_ref, m1_ref, g_ref, dinv_ref, b_ref, w_ref, out_ref):
        h = dinv_ref[...] * (m0_ref[0] + m1_ref[0] + g_ref[...]) + b_ref[...]
        h = jnp.maximum(h, 0.0)
        a = jnp.dot(h, w_ref[...], preferred_element_type=jnp.float32)
        out_ref[...] = a * dinv_ref[...]

    return pl.pallas_call(
        body,
        grid=(N // _BM,),
        in_specs=[
            *_pspec(),
            pl.BlockSpec((_BM, H), lambda i: (i, 0)),
            pl.BlockSpec((_BM, H), lambda i: (i, 0)),
            pl.BlockSpec((1, H), lambda i: (0, 0)),
            pl.BlockSpec((H, H), lambda i: (0, 0)),
        ],
        out_specs=pl.BlockSpec((_BM, H), lambda i: (i, 0)),
        out_shape=jax.ShapeDtypeStruct((N, H), jnp.float32),
    )(mp, mp, g, dinv, b, W2)


def _tc3(mp, g, dinv, b, Wfc, bfc):
    N, H = g.shape
    OUT = Wfc.shape[1]

    def body(m0_ref, m1_ref, g_ref, dinv_ref, b_ref, w_ref, bfc_ref, out_ref):
        h = dinv_ref[...] * (m0_ref[0] + m1_ref[0] + g_ref[...]) + b_ref[...]
        h = jnp.maximum(h, 0.0)
        out_ref[...] = (jnp.dot(h, w_ref[...], preferred_element_type=jnp.float32)
                        + bfc_ref[...])

    return pl.pallas_call(
        body,
        grid=(N // _BM,),
        in_specs=[
            *_pspec(),
            pl.BlockSpec((_BM, H), lambda i: (i, 0)),
            pl.BlockSpec((_BM, H), lambda i: (i, 0)),
            pl.BlockSpec((1, H), lambda i: (0, 0)),
            pl.BlockSpec((H, OUT), lambda i: (0, 0)),
            pl.BlockSpec((1, OUT), lambda i: (0, 0)),
        ],
        out_specs=pl.BlockSpec((_BM, OUT), lambda i: (i, 0)),
        out_shape=jax.ShapeDtypeStruct((N, OUT), jnp.float32),
    )(mp, mp, g, dinv, b, Wfc, bfc)


def kernel(x, edge_index, output_size, W1, b1, W2, b2, Wfc, bfc):
    N, D = x.shape
    E = edge_index.shape[1]
    H = W1.shape[1]
    OUT = Wfc.shape[1]
    assert H == _H and E % (_NW * _B) == 0 and N % _BM == 0

    edge32 = edge_index.astype(jnp.int32)
    zeros = jnp.zeros((N, H), jnp.float32)

    deg_k = _make_deg_kernel(N, E)
    msg_k = _make_msg_kernel(N, E)

    degp = deg_k(edge32, zeros)
    g1, dinv = _tc1(x, W1, degp)
    m1 = msg_k(g1, edge32, zeros)
    g2 = _tc2(m1, g1, dinv, b1.reshape(1, H), W2)
    m2 = msg_k(g2, edge32, zeros)
    return _tc3(m2, g2, dinv, b2.reshape(1, H), Wfc,
                bfc.reshape(1, OUT))
